# Initial kernel scaffold; baseline (speedup 1.0000x reference)
import jax
import jax.numpy as jnp
from jax.experimental import pallas as pl


def kernel(x, edge_index, W1l, b1, W1r, W2l, b2, W2r, Wlin, blin):
    raise NotImplementedError("write your pallas kernel here")


# speedup vs baseline: 5.9269x; 5.9269x over previous
import functools

import jax
import jax.numpy as jnp
from jax import lax
from jax.experimental import pallas as pl
from jax.experimental.pallas import tpu as pltpu
from jax.experimental.pallas import tpu_sc as plsc

N_NODES = 10000
N_PAD = 10240
N_EDGES = 320000
D = 128
NC, NS = 2, 16
NW = NC * NS
EPW = N_EDGES // NW
CH = 80
N_CH = EPW // CH
RPT = N_PAD // NS

_mesh = plsc.VectorSubcoreMesh(core_axis_name="c", subcore_axis_name="s")


def _make_agg(with_counts):
  sum_t = jax.ShapeDtypeStruct((NC, N_PAD, D), jnp.float32)
  scratch = [
      pltpu.VMEM((CH,), jnp.int32),
      pltpu.VMEM((CH,), jnp.int32),
      pltpu.VMEM((CH, D), jnp.float32),
      pltpu.VMEM_SHARED((N_PAD, D), jnp.float32),
      pltpu.SemaphoreType.DMA,
  ]
  if with_counts:
    out_type = [sum_t, jax.ShapeDtypeStruct((NW, N_PAD), jnp.float32)]
    scratch.insert(3, pltpu.VMEM((N_PAD,), jnp.float32))
  else:
    out_type = sum_t

  def body(feat, src, dst, zrow, *refs):
    if with_counts:
      sum_out, cnt_out, src_v, dst_v, rows_v, hist_v, acc_sh, sem = refs
    else:
      sum_out, src_v, dst_v, rows_v, acc_sh, sem = refs
    cid = lax.axis_index("c")
    sid = lax.axis_index("s")
    wid = sid * NC + cid
    pltpu.sync_copy(zrow, acc_sh.at[pl.ds(sid * RPT, RPT)])
    if with_counts:
      zero16 = jnp.zeros((16,), jnp.float32)

      def zstep(i, carry):
        hist_v[pl.ds(i * 16, 16)] = zero16
        return carry

      lax.fori_loop(0, N_PAD // 16, zstep, 0)
    plsc.subcore_barrier()

    base0 = wid * EPW
    one16 = jnp.ones((16,), jnp.float32)

    def step(it, carry):
      base = base0 + it * CH
      pltpu.sync_copy(src.at[pl.ds(base, CH)], src_v)
      pltpu.sync_copy(dst.at[pl.ds(base, CH)], dst_v)
      pltpu.async_copy(feat.at[src_v], rows_v, sem).wait()
      pltpu.sync_copy(rows_v, acc_sh.at[dst_v], add=True)
      if with_counts:
        for j in range(CH // 16):
          d16 = dst_v[pl.ds(j * 16, 16)]
          plsc.addupdate_scatter(hist_v, [d16], one16)
      return carry

    lax.fori_loop(0, N_CH, step, 0)
    plsc.subcore_barrier()
    pltpu.sync_copy(acc_sh.at[pl.ds(sid * RPT, RPT)],
                    sum_out.at[cid, pl.ds(sid * RPT, RPT)])
    if with_counts:
      pltpu.sync_copy(hist_v, cnt_out.at[wid])

  cp = (pltpu.CompilerParams(needs_layout_passes=False)
        if with_counts else None)
  return pl.kernel(body, out_type=out_type, mesh=_mesh,
                   scratch_types=scratch, compiler_params=cp)


_agg_with_counts = _make_agg(True)
_agg_no_counts = _make_agg(False)


def _inv_cnt(cparts):
  cnt = jnp.sum(cparts[...], axis=0)[:, None]
  return 1.0 / jnp.maximum(cnt, 1.0)


def _dense1_body(s0, s1, cparts, xr, wl, wr, b, o):
  agg = (s0[...] + s1[...]) * _inv_cnt(cparts)
  o[...] = jnp.maximum(
      jnp.dot(agg, wl[...], preferred_element_type=jnp.float32)
      + jnp.dot(xr[...], wr[...], preferred_element_type=jnp.float32)
      + b[...], 0.0)


def _dense2_body(s0, s1, cparts, hr, wl, wr, b, wab, bab, o):
  agg = (s0[...] + s1[...]) * _inv_cnt(cparts)
  h2 = jnp.maximum(
      jnp.dot(agg, wl[...], preferred_element_type=jnp.float32)
      + jnp.dot(hr[...], wr[...], preferred_element_type=jnp.float32)
      + b[...], 0.0)
  o[...] = jnp.dot(h2, wab[...], preferred_element_type=jnp.float32) + bab[...]


_RB = 1280


def _dense_specs(out_w):
  row = lambda i: (i, 0)
  full = lambda i: (0, 0)
  crow = lambda i: (0, i)
  in_specs = [
      pl.BlockSpec((_RB, D), row),
      pl.BlockSpec((_RB, D), row),
      pl.BlockSpec((NW, _RB), crow),
      pl.BlockSpec((_RB, D), row),
      pl.BlockSpec((D, D), full),
      pl.BlockSpec((D, D), full),
      pl.BlockSpec((1, D), full),
  ]
  if out_w != D:
    in_specs += [pl.BlockSpec((D, out_w), full),
                 pl.BlockSpec((1, out_w), full)]
  out_specs = pl.BlockSpec((_RB, out_w), row)
  return in_specs, out_specs


def _dense1(s0, s1, cparts, xr, wl, wr, b):
  in_specs, out_specs = _dense_specs(D)
  return pl.pallas_call(
      _dense1_body, grid=(N_PAD // _RB,), in_specs=in_specs,
      out_specs=out_specs,
      out_shape=jax.ShapeDtypeStruct((N_PAD, D), jnp.float32),
  )(s0, s1, cparts, xr, wl, wr, b)


def _dense2(s0, s1, cparts, hr, wl, wr, b, wab, bab):
  in_specs, out_specs = _dense_specs(2)
  return pl.pallas_call(
      _dense2_body, grid=(N_PAD // _RB,), in_specs=in_specs,
      out_specs=out_specs,
      out_shape=jax.ShapeDtypeStruct((N_PAD, 2), jnp.float32),
  )(s0, s1, cparts, hr, wl, wr, b, wab, bab)


def _edge_body(ab, src, dst, out, ab_v, src_v, dst_v, out_v):
  cid = lax.axis_index("c")
  sid = lax.axis_index("s")
  wid = sid * NC + cid
  pltpu.sync_copy(ab, ab_v)
  base0 = wid * EPW

  def step(it, carry):
    base = base0 + it * CH
    pltpu.sync_copy(src.at[pl.ds(base, CH)], src_v)
    pltpu.sync_copy(dst.at[pl.ds(base, CH)], dst_v)
    for j in range(CH // 16):
      s16 = src_v[pl.ds(j * 16, 16)]
      d16 = dst_v[pl.ds(j * 16, 16)]
      av = plsc.load_gather(ab_v, [s16 * 2])
      bv = plsc.load_gather(ab_v, [d16 * 2 + 1])
      out_v[pl.ds(j * 16, 16)] = av + bv
    pltpu.sync_copy(out_v, out.at[pl.ds(base, CH)])
    return carry

  lax.fori_loop(0, N_CH, step, 0)


_edge_scores = pl.kernel(
    _edge_body,
    out_type=jax.ShapeDtypeStruct((N_EDGES,), jnp.float32),
    mesh=_mesh,
    compiler_params=pltpu.CompilerParams(needs_layout_passes=False),
    scratch_types=[
        pltpu.VMEM((2 * N_PAD,), jnp.float32),
        pltpu.VMEM((CH,), jnp.int32),
        pltpu.VMEM((CH,), jnp.int32),
        pltpu.VMEM((CH,), jnp.float32),
    ],
)


def kernel(x, edge_index, W1l, b1, W1r, W2l, b2, W2r, Wlin, blin):
  src = edge_index[0].astype(jnp.int32)
  dst = edge_index[1].astype(jnp.int32)
  x_pad = jnp.zeros((N_PAD, D), jnp.float32).at[:N_NODES].set(x)
  zrow = jnp.zeros((RPT, D), jnp.float32)

  s, c = _agg_with_counts(x_pad, src, dst, zrow)
  h1 = _dense1(s[0], s[1], c, x_pad, W1l, W1r, b1.reshape(1, D))
  s2 = _agg_no_counts(h1, src, dst, zrow)
  wab = jnp.concatenate([Wlin[:D], Wlin[D:]], axis=1)
  bab = jnp.stack([blin[0], jnp.zeros((), jnp.float32)]).reshape(1, 2)
  ab = _dense2(s2[0], s2[1], c, h1, W2l, W2r, b2.reshape(1, D),
               wab, bab)
  return _edge_scores(ab.reshape(2 * N_PAD), src, dst)


# speedup vs baseline: 11.6072x; 1.9584x over previous
import functools

import jax
import jax.numpy as jnp
from jax import lax
from jax.experimental import pallas as pl
from jax.experimental.pallas import tpu as pltpu
from jax.experimental.pallas import tpu_sc as plsc

N_NODES = 10000
N_PAD = 10240
N_EDGES = 320000
D = 128
NC, NS = 2, 16
NW = NC * NS
EPW = N_EDGES // NW
CH = 80
N_CH = EPW // CH
RPT = N_PAD // NS

_mesh = plsc.VectorSubcoreMesh(core_axis_name="c", subcore_axis_name="s")


def _make_agg():
  sum_t = jax.ShapeDtypeStruct((NC, N_PAD, D), jnp.float32)
  scratch = [
      pltpu.VMEM((EPW,), jnp.int32),
      pltpu.VMEM((N_CH, CH), jnp.int32),
      pltpu.VMEM((CH, D), jnp.float32),
      pltpu.VMEM((CH, D), jnp.float32),
      pltpu.VMEM_SHARED((N_PAD, D), jnp.float32),
      pltpu.SemaphoreType.DMA,
      pltpu.SemaphoreType.DMA,
  ]

  def body(feat, src2, dst3, zrow, sum_out, src_a, dst_a, rows0, rows1,
           acc_sh, s0, s1):
    rows = (rows0, rows1)
    sems = (s0, s1)
    cid = lax.axis_index("c")
    sid = lax.axis_index("s")
    wid = sid * NC + cid
    pltpu.sync_copy(zrow, acc_sh.at[pl.ds(sid * RPT, RPT)])
    pltpu.sync_copy(src2.at[wid], src_a)
    pltpu.sync_copy(dst3.at[wid], dst_a)
    plsc.subcore_barrier()

    pltpu.async_copy(feat.at[src_a.at[pl.ds(0, CH)]], rows0, s0)

    def pair(k, carry):
      for b in range(2):
        c = 2 * k + b
        pltpu.make_async_copy(feat.at[pl.ds(0, CH)], rows[b], sems[b]).wait()
        pltpu.async_copy(feat.at[src_a.at[pl.ds((c + 1) * CH, CH)]],
                         rows[b ^ 1], sems[b ^ 1])
        pltpu.sync_copy(rows[b], acc_sh.at[dst_a.at[c]], add=True)
      return carry

    lax.fori_loop(0, (N_CH - 1) // 2, pair, 0)
    pltpu.make_async_copy(feat.at[pl.ds(0, CH)], rows0, s0).wait()
    pltpu.sync_copy(rows0, acc_sh.at[dst_a.at[N_CH - 1]], add=True)

    plsc.subcore_barrier()
    pltpu.sync_copy(acc_sh.at[pl.ds(sid * RPT, RPT)],
                    sum_out.at[cid, pl.ds(sid * RPT, RPT)])

  return pl.kernel(
      body, out_type=sum_t, mesh=_mesh, scratch_types=scratch,
      compiler_params=pltpu.CompilerParams(needs_layout_passes=False))


def _hist_body(dst3, cnt_out, dst_a, hist_v):
  cid = lax.axis_index("c")
  sid = lax.axis_index("s")
  wid = sid * NC + cid
  pltpu.sync_copy(dst3.at[wid], dst_a)
  zero16 = jnp.zeros((16,), jnp.float32)

  def zstep(i, carry):
    hist_v[pl.ds(i * 16, 16)] = zero16
    return carry

  lax.fori_loop(0, N_PAD // 16, zstep, 0)
  one16 = jnp.ones((16,), jnp.float32)

  def step(c, carry):
    for j in range(CH // 16):
      d16 = dst_a[c, pl.ds(j * 16, 16)]
      plsc.addupdate_scatter(hist_v, [d16], one16)
    return carry

  lax.fori_loop(0, N_CH, step, 0)
  pltpu.sync_copy(hist_v, cnt_out.at[wid])


_degree_hist = pl.kernel(
    _hist_body,
    out_type=jax.ShapeDtypeStruct((NW, N_PAD), jnp.float32),
    mesh=_mesh,
    compiler_params=pltpu.CompilerParams(needs_layout_passes=False),
    scratch_types=[
        pltpu.VMEM((N_CH, CH), jnp.int32),
        pltpu.VMEM((N_PAD,), jnp.float32),
    ],
)


_segment_sum = _make_agg()


def _inv_cnt(cparts):
  cnt = jnp.sum(cparts[...], axis=0)[:, None]
  return 1.0 / jnp.maximum(cnt, 1.0)


def _dense1_body(s0, s1, cparts, xr, wl, wr, b, o):
  agg = (s0[...] + s1[...]) * _inv_cnt(cparts)
  o[...] = jnp.maximum(
      jnp.dot(agg, wl[...], preferred_element_type=jnp.float32)
      + jnp.dot(xr[...], wr[...], preferred_element_type=jnp.float32)
      + b[...], 0.0)


def _dense2_body(s0, s1, cparts, hr, wl, wr, b, wab, bab, o):
  agg = (s0[...] + s1[...]) * _inv_cnt(cparts)
  h2 = jnp.maximum(
      jnp.dot(agg, wl[...], preferred_element_type=jnp.float32)
      + jnp.dot(hr[...], wr[...], preferred_element_type=jnp.float32)
      + b[...], 0.0)
  o[...] = jnp.dot(h2, wab[...], preferred_element_type=jnp.float32) + bab[...]


_RB = 1280


def _dense_specs(out_w):
  row = lambda i: (i, 0)
  full = lambda i: (0, 0)
  crow = lambda i: (0, i)
  in_specs = [
      pl.BlockSpec((_RB, D), row),
      pl.BlockSpec((_RB, D), row),
      pl.BlockSpec((NW, _RB), crow),
      pl.BlockSpec((_RB, D), row),
      pl.BlockSpec((D, D), full),
      pl.BlockSpec((D, D), full),
      pl.BlockSpec((1, D), full),
  ]
  if out_w != D:
    in_specs += [pl.BlockSpec((D, out_w), full),
                 pl.BlockSpec((1, out_w), full)]
  out_specs = pl.BlockSpec((_RB, out_w), row)
  return in_specs, out_specs


def _dense1(s0, s1, cparts, xr, wl, wr, b):
  in_specs, out_specs = _dense_specs(D)
  return pl.pallas_call(
      _dense1_body, grid=(N_PAD // _RB,), in_specs=in_specs,
      out_specs=out_specs,
      out_shape=jax.ShapeDtypeStruct((N_PAD, D), jnp.float32),
  )(s0, s1, cparts, xr, wl, wr, b)


def _dense2(s0, s1, cparts, hr, wl, wr, b, wab, bab):
  in_specs, out_specs = _dense_specs(2)
  return pl.pallas_call(
      _dense2_body, grid=(N_PAD // _RB,), in_specs=in_specs,
      out_specs=out_specs,
      out_shape=jax.ShapeDtypeStruct((N_PAD, 2), jnp.float32),
  )(s0, s1, cparts, hr, wl, wr, b, wab, bab)


def _edge_body(ab, src3, dst3, out3, ab_v, src_a, dst_a, out_a):
  cid = lax.axis_index("c")
  sid = lax.axis_index("s")
  wid = sid * NC + cid
  pltpu.sync_copy(ab, ab_v)
  pltpu.sync_copy(src3.at[wid], src_a)
  pltpu.sync_copy(dst3.at[wid], dst_a)

  def step(c, carry):
    for j in range(CH // 16):
      s16 = src_a[c, pl.ds(j * 16, 16)]
      d16 = dst_a[c, pl.ds(j * 16, 16)]
      av = plsc.load_gather(ab_v, [s16 * 2])
      bv = plsc.load_gather(ab_v, [d16 * 2 + 1])
      out_a[c, pl.ds(j * 16, 16)] = av + bv
    return carry

  lax.fori_loop(0, N_CH, step, 0)
  pltpu.sync_copy(out_a, out3.at[wid])


_edge_scores = pl.kernel(
    _edge_body,
    out_type=jax.ShapeDtypeStruct((NW, N_CH, CH), jnp.float32),
    mesh=_mesh,
    compiler_params=pltpu.CompilerParams(needs_layout_passes=False),
    scratch_types=[
        pltpu.VMEM((2 * N_PAD,), jnp.float32),
        pltpu.VMEM((N_CH, CH), jnp.int32),
        pltpu.VMEM((N_CH, CH), jnp.int32),
        pltpu.VMEM((N_CH, CH), jnp.float32),
    ],
)


def kernel(x, edge_index, W1l, b1, W1r, W2l, b2, W2r, Wlin, blin):
  src3 = edge_index[0].astype(jnp.int32).reshape(NW, N_CH, CH)
  dst3 = edge_index[1].astype(jnp.int32).reshape(NW, N_CH, CH)
  x_pad = jnp.zeros((N_PAD, D), jnp.float32).at[:N_NODES].set(x)
  zrow = jnp.zeros((RPT, D), jnp.float32)

  src2 = src3.reshape(NW, EPW)
  c = _degree_hist(dst3)
  s = _segment_sum(x_pad, src2, dst3, zrow)
  h1 = _dense1(s[0], s[1], c, x_pad, W1l, W1r, b1.reshape(1, D))
  s2 = _segment_sum(h1, src2, dst3, zrow)
  wab = jnp.concatenate([Wlin[:D], Wlin[D:]], axis=1)
  bab = jnp.stack([blin[0], jnp.zeros((), jnp.float32)]).reshape(1, 2)
  ab = _dense2(s2[0], s2[1], c, h1, W2l, W2r, b2.reshape(1, D),
               wab, bab)
  return _edge_scores(ab.reshape(2 * N_PAD), src3, dst3).reshape(N_EDGES)


# speedup vs baseline: 15.7102x; 1.3535x over previous
import functools

import jax
import jax.numpy as jnp
from jax import lax
from jax.experimental import pallas as pl
from jax.experimental.pallas import tpu as pltpu
from jax.experimental.pallas import tpu_sc as plsc

N_NODES = 10000
N_PAD = 10240
N_EDGES = 320000
D = 128
NC, NS = 2, 16
NW = NC * NS
EPW = N_EDGES // NW
CH = 80
N_CH = EPW // CH
RPT = N_PAD // NS

_mesh = plsc.VectorSubcoreMesh(core_axis_name="c", subcore_axis_name="s")


BL = 8
NB = 16
N_CHP = NB * BL
NBUF = 4


def _make_agg():
  sum_t = jax.ShapeDtypeStruct((NC, N_PAD, D), jnp.float32)
  scratch = [
      pltpu.VMEM((2 * BL * CH,), jnp.int32),
      pltpu.VMEM((2, BL, CH), jnp.int32),
  ] + [pltpu.VMEM((CH, D), jnp.float32) for _ in range(NBUF)] + [
      pltpu.VMEM_SHARED((N_PAD, D), jnp.float32),
  ] + [pltpu.SemaphoreType.DMA for _ in range(2 * NBUF + 1)]

  def body(feat, src2, dst3, zrow, sum_out, src_b, dst_b, *refs):
    rows = refs[0:NBUF]
    acc_sh = refs[NBUF]
    gsem = refs[NBUF + 1:2 * NBUF + 1]
    ssem = refs[2 * NBUF + 1:3 * NBUF + 1]
    rsem = refs[3 * NBUF + 1]
    cid = lax.axis_index("c")
    sid = lax.axis_index("s")
    wid = sid * NC + cid
    pltpu.sync_copy(zrow, acc_sh.at[pl.ds(sid * RPT, RPT)])

    def refill(blk, slot):
      pltpu.async_copy(src2.at[wid, pl.ds(blk * BL * CH, BL * CH)],
                       src_b.at[pl.ds(slot * BL * CH, BL * CH)], rsem)
      pltpu.async_copy(dst3.at[wid, pl.ds(blk * BL, BL)], dst_b.at[slot], rsem)

    def drain_refill(slot):
      pltpu.make_async_copy(src2.at[0, pl.ds(0, BL * CH)],
                            src_b.at[pl.ds(slot * BL * CH, BL * CH)],
                            rsem).wait()
      pltpu.make_async_copy(dst3.at[0, pl.ds(0, BL)], dst_b.at[slot],
                            rsem).wait()

    def issue_gather(c, b):
      off = (c // BL) % 2 * (BL * CH) + (c % BL) * CH
      pltpu.async_copy(feat.at[src_b.at[pl.ds(off, CH)]], rows[b], gsem[b])

    def drain(buf, sem):
      pltpu.make_async_copy(feat.at[pl.ds(0, CH)], buf, sem).wait()

    refill(0, 0)
    drain_refill(0)
    refill(1, 1)
    plsc.subcore_barrier()
    issue_gather(0, 0)
    issue_gather(1, 1)

    def quad(k, carry):
      for b in range(NBUF):
        c = NBUF * k + b
        cn = c + 2
        bn = (b + 2) % NBUF

        @pl.when(c >= 2)
        def _free():
          drain(rows[bn], ssem[bn])

        @pl.when(cn < N_CH)
        def _issue():
          @pl.when(cn % BL == 0)
          def _rwait():
            drain_refill((cn // BL) % 2)

          @pl.when((cn % BL == 4) & (cn >= BL + 4)
                   & (cn // BL + 1 <= NB - 1))
          def _rnext():
            refill(cn // BL + 1, (cn // BL + 1) % 2)

          issue_gather(cn, bn)

        drain(rows[b], gsem[b])
        pltpu.async_copy(rows[b],
                         acc_sh.at[dst_b.at[(c // BL) % 2, c % BL]],
                         ssem[b], add=True)
      return carry

    lax.fori_loop(0, (N_CH - 1) // NBUF, quad, 0)
    c = N_CH - 1
    drain(rows[c % NBUF], gsem[c % NBUF])
    pltpu.async_copy(rows[c % NBUF],
                     acc_sh.at[dst_b.at[(c // BL) % 2, c % BL]],
                     ssem[c % NBUF], add=True)
    for cc in (N_CH - 3, N_CH - 2, N_CH - 1):
      drain(rows[cc % NBUF], ssem[cc % NBUF])

    plsc.subcore_barrier()
    pltpu.sync_copy(acc_sh.at[pl.ds(sid * RPT, RPT)],
                    sum_out.at[cid, pl.ds(sid * RPT, RPT)])

  return pl.kernel(
      body, out_type=sum_t, mesh=_mesh, scratch_types=scratch,
      compiler_params=pltpu.CompilerParams(needs_layout_passes=False))


def _hist_body(dst3, cnt_out, dst_a, hist_v):
  cid = lax.axis_index("c")
  sid = lax.axis_index("s")
  wid = sid * NC + cid
  pltpu.sync_copy(dst3.at[wid], dst_a)
  zero16 = jnp.zeros((16,), jnp.float32)

  def zstep(i, carry):
    hist_v[pl.ds(i * 16, 16)] = zero16
    return carry

  lax.fori_loop(0, N_PAD // 16, zstep, 0)
  one16 = jnp.ones((16,), jnp.float32)

  def step(c, carry):
    for j in range(CH // 16):
      d16 = dst_a[c, pl.ds(j * 16, 16)]
      plsc.addupdate_scatter(hist_v, [d16], one16)
    return carry

  lax.fori_loop(0, N_CH, step, 0)
  pltpu.sync_copy(hist_v, cnt_out.at[wid])


_degree_hist = pl.kernel(
    _hist_body,
    out_type=jax.ShapeDtypeStruct((NW, N_PAD), jnp.float32),
    mesh=_mesh,
    compiler_params=pltpu.CompilerParams(needs_layout_passes=False),
    scratch_types=[
        pltpu.VMEM((N_CH, CH), jnp.int32),
        pltpu.VMEM((N_PAD,), jnp.float32),
    ],
)


_segment_sum = _make_agg()


def _inv_cnt(cparts):
  cnt = jnp.sum(cparts[...], axis=0)[:, None]
  return 1.0 / jnp.maximum(cnt, 1.0)


def _dense1_body(s0, s1, cparts, xr, wl, wr, b, o):
  agg = (s0[...] + s1[...]) * _inv_cnt(cparts)
  o[...] = jnp.maximum(
      jnp.dot(agg, wl[...], preferred_element_type=jnp.float32)
      + jnp.dot(xr[...], wr[...], preferred_element_type=jnp.float32)
      + b[...], 0.0)


def _dense2_body(s0, s1, cparts, hr, wl, wr, b, wab, bab, o):
  agg = (s0[...] + s1[...]) * _inv_cnt(cparts)
  h2 = jnp.maximum(
      jnp.dot(agg, wl[...], preferred_element_type=jnp.float32)
      + jnp.dot(hr[...], wr[...], preferred_element_type=jnp.float32)
      + b[...], 0.0)
  o[...] = jnp.dot(h2, wab[...], preferred_element_type=jnp.float32) + bab[...]


_RB = 1280


def _dense_specs(out_w):
  row = lambda i: (i, 0)
  full = lambda i: (0, 0)
  crow = lambda i: (0, i)
  in_specs = [
      pl.BlockSpec((_RB, D), row),
      pl.BlockSpec((_RB, D), row),
      pl.BlockSpec((NW, _RB), crow),
      pl.BlockSpec((_RB, D), row),
      pl.BlockSpec((D, D), full),
      pl.BlockSpec((D, D), full),
      pl.BlockSpec((1, D), full),
  ]
  if out_w != D:
    in_specs += [pl.BlockSpec((D, out_w), full),
                 pl.BlockSpec((1, out_w), full)]
  out_specs = pl.BlockSpec((_RB, out_w), row)
  return in_specs, out_specs


def _dense1(s0, s1, cparts, xr, wl, wr, b):
  in_specs, out_specs = _dense_specs(D)
  return pl.pallas_call(
      _dense1_body, grid=(N_PAD // _RB,), in_specs=in_specs,
      out_specs=out_specs,
      out_shape=jax.ShapeDtypeStruct((N_PAD, D), jnp.float32),
  )(s0, s1, cparts, xr, wl, wr, b)


def _dense2(s0, s1, cparts, hr, wl, wr, b, wab, bab):
  in_specs, out_specs = _dense_specs(2)
  return pl.pallas_call(
      _dense2_body, grid=(N_PAD // _RB,), in_specs=in_specs,
      out_specs=out_specs,
      out_shape=jax.ShapeDtypeStruct((N_PAD, 2), jnp.float32),
  )(s0, s1, cparts, hr, wl, wr, b, wab, bab)


def _edge_body(ab, src3, dst3, out3, ab_v, src_a, dst_a, out_a):
  cid = lax.axis_index("c")
  sid = lax.axis_index("s")
  wid = sid * NC + cid
  pltpu.sync_copy(ab, ab_v)
  pltpu.sync_copy(src3.at[wid], src_a)
  pltpu.sync_copy(dst3.at[wid], dst_a)

  def step(c, carry):
    for j in range(CH // 16):
      s16 = src_a[c, pl.ds(j * 16, 16)]
      d16 = dst_a[c, pl.ds(j * 16, 16)]
      av = plsc.load_gather(ab_v, [s16 * 2])
      bv = plsc.load_gather(ab_v, [d16 * 2 + 1])
      out_a[c, pl.ds(j * 16, 16)] = av + bv
    return carry

  lax.fori_loop(0, N_CH, step, 0)
  pltpu.sync_copy(out_a, out3.at[wid])


_edge_scores = pl.kernel(
    _edge_body,
    out_type=jax.ShapeDtypeStruct((NW, N_CH, CH), jnp.float32),
    mesh=_mesh,
    compiler_params=pltpu.CompilerParams(needs_layout_passes=False),
    scratch_types=[
        pltpu.VMEM((2 * N_PAD,), jnp.float32),
        pltpu.VMEM((N_CH, CH), jnp.int32),
        pltpu.VMEM((N_CH, CH), jnp.int32),
        pltpu.VMEM((N_CH, CH), jnp.float32),
    ],
)


def kernel(x, edge_index, W1l, b1, W1r, W2l, b2, W2r, Wlin, blin):
  src3 = edge_index[0].astype(jnp.int32).reshape(NW, N_CH, CH)
  dst3 = edge_index[1].astype(jnp.int32).reshape(NW, N_CH, CH)
  x_pad = jnp.zeros((N_PAD, D), jnp.float32).at[:N_NODES].set(x)
  zrow = jnp.zeros((RPT, D), jnp.float32)

  pad_ch = N_CHP - N_CH
  src2p = jnp.pad(src3.reshape(NW, EPW), ((0, 0), (0, pad_ch * CH)))
  dst3p = jnp.pad(dst3, ((0, 0), (0, pad_ch), (0, 0)))
  c = _degree_hist(dst3)
  s = _segment_sum(x_pad, src2p, dst3p, zrow)
  h1 = _dense1(s[0], s[1], c, x_pad, W1l, W1r, b1.reshape(1, D))
  s2 = _segment_sum(h1, src2p, dst3p, zrow)
  wab = jnp.concatenate([Wlin[:D], Wlin[D:]], axis=1)
  bab = jnp.stack([blin[0], jnp.zeros((), jnp.float32)]).reshape(1, 2)
  ab = _dense2(s2[0], s2[1], c, h1, W2l, W2r, b2.reshape(1, D),
               wab, bab)
  return _edge_scores(ab.reshape(2 * N_PAD), src3, dst3).reshape(N_EDGES)


# speedup vs baseline: 15.7735x; 1.0040x over previous
import jax
import jax.numpy as jnp
from jax import lax
from jax.experimental import pallas as pl
from jax.experimental.pallas import tpu as pltpu
from jax.experimental.pallas import tpu_sc as plsc

N_NODES = 10000
N_PAD = 10240
N_EDGES = 320000
D = 128
NC, NS = 2, 16
NW = NC * NS
EPW = N_EDGES // NW
CH = 80
N_CH = EPW // CH
RPT = N_PAD // NS

_mesh = plsc.VectorSubcoreMesh(core_axis_name="c", subcore_axis_name="s")


BL = 8
NB = 16
N_CHP = NB * BL
NBUF = 4


def _make_agg(with_counts):
  sum_t = jax.ShapeDtypeStruct((NC, N_PAD, D), jnp.float32)
  if with_counts:
    out_type = [sum_t, jax.ShapeDtypeStruct((NW, N_PAD), jnp.float32)]
  else:
    out_type = sum_t
  scratch = [
      pltpu.VMEM_SHARED((N_PAD, D), jnp.float32),
  ] + [pltpu.SemaphoreType.DMA for _ in range(2 * NBUF + 1)]

  def body(feat, src2, dst3, zrow, *refs):
    if with_counts:
      sum_out, cnt_out = refs[0], refs[1]
      refs = refs[2:]
    else:
      sum_out = refs[0]
      refs = refs[1:]
    acc_sh = refs[0]
    gsem = refs[1:NBUF + 1]
    ssem = refs[NBUF + 1:2 * NBUF + 1]
    rsem = refs[2 * NBUF + 1]
    cid = lax.axis_index("c")
    sid = lax.axis_index("s")
    wid = sid * NC + cid
    pltpu.sync_copy(zrow, acc_sh.at[pl.ds(sid * RPT, RPT)])

    if with_counts:
      def hist_phase(dst_all, hist_v):
        pltpu.sync_copy(dst3.at[wid], dst_all)
        zero16 = jnp.zeros((16,), jnp.float32)

        def zstep(i, carry):
          hist_v[pl.ds(i * 16, 16)] = zero16
          return carry

        lax.fori_loop(0, N_PAD // 16, zstep, 0)
        one16 = jnp.ones((16,), jnp.float32)

        def step(ch, carry):
          for j in range(CH // 16):
            d16 = dst_all[ch, pl.ds(j * 16, 16)]
            plsc.addupdate_scatter(hist_v, [d16], one16)
          return carry

        lax.fori_loop(0, N_CH, step, 0)
        pltpu.sync_copy(hist_v, cnt_out.at[wid])

      pl.run_scoped(hist_phase,
                    pltpu.VMEM((N_CHP, CH), jnp.int32),
                    pltpu.VMEM((N_PAD,), jnp.float32))

    def agg_phase(src_b, dst_b, *rows):
      def refill(blk, slot):
        pltpu.async_copy(src2.at[wid, pl.ds(blk * BL * CH, BL * CH)],
                         src_b.at[pl.ds(slot * BL * CH, BL * CH)], rsem)
        pltpu.async_copy(dst3.at[wid, pl.ds(blk * BL, BL)], dst_b.at[slot],
                         rsem)

      def drain_refill(slot):
        pltpu.make_async_copy(src2.at[0, pl.ds(0, BL * CH)],
                              src_b.at[pl.ds(slot * BL * CH, BL * CH)],
                              rsem).wait()
        pltpu.make_async_copy(dst3.at[0, pl.ds(0, BL)], dst_b.at[slot],
                              rsem).wait()

      def issue_gather(c, b):
        off = (c // BL) % 2 * (BL * CH) + (c % BL) * CH
        pltpu.async_copy(feat.at[src_b.at[pl.ds(off, CH)]], rows[b], gsem[b])

      def drain(buf, sem):
        pltpu.make_async_copy(feat.at[pl.ds(0, CH)], buf, sem).wait()

      refill(0, 0)
      drain_refill(0)
      refill(1, 1)
      plsc.subcore_barrier()
      issue_gather(0, 0)
      issue_gather(1, 1)

      def quad(k, carry):
        for b in range(NBUF):
          c = NBUF * k + b
          cn = c + 2
          bn = (b + 2) % NBUF

          @pl.when(c >= 2)
          def _free():
            drain(rows[bn], ssem[bn])

          @pl.when(cn < N_CH)
          def _issue():
            @pl.when(cn % BL == 0)
            def _rwait():
              drain_refill((cn // BL) % 2)

            @pl.when((cn % BL == 4) & (cn >= BL + 4)
                     & (cn // BL + 1 <= NB - 1))
            def _rnext():
              refill(cn // BL + 1, (cn // BL + 1) % 2)

            issue_gather(cn, bn)

          drain(rows[b], gsem[b])
          pltpu.async_copy(rows[b],
                           acc_sh.at[dst_b.at[(c // BL) % 2, c % BL]],
                           ssem[b], add=True)
        return carry

      lax.fori_loop(0, (N_CH - 1) // NBUF, quad, 0)
      c = N_CH - 1
      drain(rows[c % NBUF], gsem[c % NBUF])
      pltpu.async_copy(rows[c % NBUF],
                       acc_sh.at[dst_b.at[(c // BL) % 2, c % BL]],
                       ssem[c % NBUF], add=True)
      for cc in (N_CH - 3, N_CH - 2, N_CH - 1):
        drain(rows[cc % NBUF], ssem[cc % NBUF])

    pl.run_scoped(agg_phase,
                  pltpu.VMEM((2 * BL * CH,), jnp.int32),
                  pltpu.VMEM((2, BL, CH), jnp.int32),
                  *[pltpu.VMEM((CH, D), jnp.float32) for _ in range(NBUF)])

    plsc.subcore_barrier()
    pltpu.sync_copy(acc_sh.at[pl.ds(sid * RPT, RPT)],
                    sum_out.at[cid, pl.ds(sid * RPT, RPT)])

  return pl.kernel(
      body, out_type=out_type, mesh=_mesh, scratch_types=scratch,
      compiler_params=pltpu.CompilerParams(needs_layout_passes=False))


_agg_counts = _make_agg(True)
_agg_plain = _make_agg(False)


def _inv_cnt(cparts):
  cnt = jnp.sum(cparts[...], axis=0)[:, None]
  return 1.0 / jnp.maximum(cnt, 1.0)


def _dense1_body(s0, s1, cparts, xr, wl, wr, b, o):
  agg = (s0[...] + s1[...]) * _inv_cnt(cparts)
  o[...] = jnp.maximum(
      jnp.dot(agg, wl[...], preferred_element_type=jnp.float32)
      + jnp.dot(xr[...], wr[...], preferred_element_type=jnp.float32)
      + b[...], 0.0)


def _dense2_body(s0, s1, cparts, hr, wl, wr, b, wab, bab, o):
  agg = (s0[...] + s1[...]) * _inv_cnt(cparts)
  h2 = jnp.maximum(
      jnp.dot(agg, wl[...], preferred_element_type=jnp.float32)
      + jnp.dot(hr[...], wr[...], preferred_element_type=jnp.float32)
      + b[...], 0.0)
  o[...] = jnp.dot(h2, wab[...], preferred_element_type=jnp.float32) + bab[...]


_RB = 1280


def _dense_specs(out_w):
  row = lambda i: (i, 0)
  full = lambda i: (0, 0)
  crow = lambda i: (0, i)
  in_specs = [
      pl.BlockSpec((_RB, D), row),
      pl.BlockSpec((_RB, D), row),
      pl.BlockSpec((NW, _RB), crow),
      pl.BlockSpec((_RB, D), row),
      pl.BlockSpec((D, D), full),
      pl.BlockSpec((D, D), full),
      pl.BlockSpec((1, D), full),
  ]
  if out_w != D:
    in_specs += [pl.BlockSpec((D, out_w), full),
                 pl.BlockSpec((1, out_w), full)]
  out_specs = pl.BlockSpec((_RB, out_w), row)
  return in_specs, out_specs


def _dense1(s0, s1, cparts, xr, wl, wr, b):
  in_specs, out_specs = _dense_specs(D)
  return pl.pallas_call(
      _dense1_body, grid=(N_PAD // _RB,), in_specs=in_specs,
      out_specs=out_specs,
      out_shape=jax.ShapeDtypeStruct((N_PAD, D), jnp.float32),
  )(s0, s1, cparts, xr, wl, wr, b)


def _dense2(s0, s1, cparts, hr, wl, wr, b, wab, bab):
  in_specs, out_specs = _dense_specs(2)
  return pl.pallas_call(
      _dense2_body, grid=(N_PAD // _RB,), in_specs=in_specs,
      out_specs=out_specs,
      out_shape=jax.ShapeDtypeStruct((N_PAD, 2), jnp.float32),
  )(s0, s1, cparts, hr, wl, wr, b, wab, bab)


def _edge_body(ab, src3, dst3, out3, ab_v, src_a, dst_a, out_a):
  cid = lax.axis_index("c")
  sid = lax.axis_index("s")
  wid = sid * NC + cid
  pltpu.sync_copy(ab, ab_v)
  pltpu.sync_copy(src3.at[wid], src_a)
  pltpu.sync_copy(dst3.at[wid], dst_a)

  def step(c, carry):
    for j in range(CH // 16):
      s16 = src_a[c, pl.ds(j * 16, 16)]
      d16 = dst_a[c, pl.ds(j * 16, 16)]
      av = plsc.load_gather(ab_v, [s16 * 2])
      bv = plsc.load_gather(ab_v, [d16 * 2 + 1])
      out_a[c, pl.ds(j * 16, 16)] = av + bv
    return carry

  lax.fori_loop(0, N_CH, step, 0)
  pltpu.sync_copy(out_a, out3.at[wid])


_edge_scores = pl.kernel(
    _edge_body,
    out_type=jax.ShapeDtypeStruct((NW, N_CH, CH), jnp.float32),
    mesh=_mesh,
    compiler_params=pltpu.CompilerParams(needs_layout_passes=False),
    scratch_types=[
        pltpu.VMEM((2 * N_PAD,), jnp.float32),
        pltpu.VMEM((N_CH, CH), jnp.int32),
        pltpu.VMEM((N_CH, CH), jnp.int32),
        pltpu.VMEM((N_CH, CH), jnp.float32),
    ],
)


def kernel(x, edge_index, W1l, b1, W1r, W2l, b2, W2r, Wlin, blin):
  src3 = edge_index[0].astype(jnp.int32).reshape(NW, N_CH, CH)
  dst3 = edge_index[1].astype(jnp.int32).reshape(NW, N_CH, CH)
  x_pad = jnp.zeros((N_PAD, D), jnp.float32).at[:N_NODES].set(x)
  zrow = jnp.zeros((RPT, D), jnp.float32)

  pad_ch = N_CHP - N_CH
  src2p = jnp.pad(src3.reshape(NW, EPW), ((0, 0), (0, pad_ch * CH)))
  dst3p = jnp.pad(dst3, ((0, 0), (0, pad_ch), (0, 0)))
  s, c = _agg_counts(x_pad, src2p, dst3p, zrow)
  h1 = _dense1(s[0], s[1], c, x_pad, W1l, W1r, b1.reshape(1, D))
  s2 = _agg_plain(h1, src2p, dst3p, zrow)
  wab = jnp.concatenate([Wlin[:D], Wlin[D:]], axis=1)
  bab = jnp.stack([blin[0], jnp.zeros((), jnp.float32)]).reshape(1, 2)
  ab = _dense2(s2[0], s2[1], c, h1, W2l, W2r, b2.reshape(1, D),
               wab, bab)
  return _edge_scores(ab.reshape(2 * N_PAD), src3, dst3).reshape(N_EDGES)
